# baseline (speedup 1.0000x reference)
import functools

import numpy as np
import jax
import jax.numpy as jnp
from jax import lax
from jax.experimental import pallas as pl
from jax.experimental.pallas import tpu as pltpu
from jax.experimental.pallas import tpu_sc as plsc

_NUM_TYPES = 4
_NUM_SPLINES = 16
_SPLINE_SPAN = 12
_NUM_CHANNELS = 64
_N_NODES = 10000
_N_EDGES = 160000


_NC = 2
_NS = 16
_NW = _NC * _NS
_NTILES = _N_EDGES // 128
_WTILES = 40
_WEDGES = _WTILES * 128
_WVECS = _WEDGES // 16


def _classes_body(types_hbm, ei_hbm, cls_hbm, table_v, ei_v, c_v):
    wid = lax.axis_index("s") * _NC + lax.axis_index("c")
    t0 = (wid * _NTILES) // _NW
    base = t0 * 128
    pltpu.sync_copy(types_hbm, table_v)
    pltpu.sync_copy(ei_hbm.at[:, pl.ds(base, _WEDGES)], ei_v)

    def body(i, carry):
        off = i * 16
        t0v = plsc.load_gather(table_v, [ei_v[0, pl.ds(off, 16)]])
        t1v = plsc.load_gather(table_v, [ei_v[1, pl.ds(off, 16)]])
        c_v[pl.ds(off, 16)] = t0v * _NUM_TYPES + t1v
        return carry

    lax.fori_loop(0, _WVECS, body, 0)
    pltpu.sync_copy(c_v, cls_hbm.at[pl.ds(base, _WEDGES)])


def _make_classes_call():
    mesh = plsc.VectorSubcoreMesh(core_axis_name="c", subcore_axis_name="s")
    return pl.kernel(
        _classes_body,
        mesh=mesh,
        compiler_params=pltpu.CompilerParams(needs_layout_passes=False),
        out_type=jax.ShapeDtypeStruct((_N_EDGES,), jnp.int32),
        scratch_types=[
            pltpu.VMEM((_N_NODES,), jnp.int32),
            pltpu.VMEM((2, _WEDGES), jnp.int32),
            pltpu.VMEM((_WEDGES,), jnp.int32),
        ],
    )


_B = 3200
_NB = _N_EDGES // _B
_NSC = _NUM_TYPES * _NUM_TYPES * _NUM_SPLINES
_WIDTH = _SPLINE_SPAN / _NUM_SPLINES

_FREQ = np.float32(np.pi / _WIDTH)

_J = np.arange(_NSC)
_SJ = _J % _NUM_SPLINES
_KJ = _J // _NUM_SPLINES
_CJ = _SJ / (_NUM_SPLINES - 1.0)
_CLS_BLOCK = 2.0 * (_KJ[:, None] == np.arange(_NUM_TYPES * _NUM_TYPES)[None, :])
_QN = np.arange(32)
_QM = (_QN - 1) >> 1
_LO = _QM - 10
_HI = _QM + 11 + ((_QN - 1) & 1)
_WIN_BLOCK = 2.0 * ((_SJ[:, None] >= _LO[None, :]) & (_SJ[:, None] <= _HI[None, :]))
_MF = np.concatenate([
    (0.5 * np.cos(np.pi * _CJ / _WIDTH))[:, None],
    (0.5 * np.sin(np.pi * _CJ / _WIDTH))[:, None],
    np.full((_NSC, 1), 0.5 - 4.0),
    _CLS_BLOCK,
    _WIN_BLOCK,
], axis=1).astype(np.float32)


def _embed_body(nl_ref, cls_ref, mf_ref, w_ref, out_ref):
    off = pl.multiple_of(pl.program_id(0) * _B, _B)
    x = nl_ref[pl.ds(off, _B)].reshape(1, _B)
    cls = cls_ref[pl.ds(off, _B)].reshape(1, _B)
    a = x * _FREQ
    ohc = (lax.broadcasted_iota(jnp.int32, (16, _B), 0) == cls
           ).astype(jnp.bfloat16)
    qn = jnp.floor(x * 30.0 - 0.5).astype(jnp.int32) + 1
    ohw = (lax.broadcasted_iota(jnp.int32, (32, _B), 0) == qn
           ).astype(jnp.bfloat16)
    rows = jnp.concatenate(
        [jnp.cos(a).astype(jnp.bfloat16), jnp.sin(a).astype(jnp.bfloat16),
         jnp.ones((1, _B), jnp.bfloat16), ohc, ohw], axis=0)
    combined = jnp.maximum(
        jnp.dot(mf_ref[...], rows, preferred_element_type=jnp.float32),
        0.0).astype(jnp.bfloat16)
    out_ref[...] = lax.dot_general(
        w_ref[...], combined, (((0,), (0,)), ((), ())),
        preferred_element_type=jnp.float32)


def _make_embed_call():
    return pl.pallas_call(
        _embed_body,
        grid=(_NB,),
        in_specs=[
            pl.BlockSpec((_N_EDGES,), lambda i: (0,)),
            pl.BlockSpec((_N_EDGES,), lambda i: (0,)),
            pl.BlockSpec((_NSC, 51), lambda i: (0, 0)),
            pl.BlockSpec((_NSC, _NUM_CHANNELS), lambda i: (0, 0)),
        ],
        out_specs=pl.BlockSpec((_NUM_CHANNELS, _B), lambda i: (0, i)),
        out_shape=jax.ShapeDtypeStruct((_NUM_CHANNELS, _N_EDGES), jnp.float32),
    )


def kernel(norm_length, atom_types, edge_index, class_embed_weight):
    at = atom_types.reshape(-1).astype(jnp.int32)
    classes = _make_classes_call()(at, edge_index.astype(jnp.int32))
    w = class_embed_weight.reshape(_NSC, _NUM_CHANNELS).astype(jnp.bfloat16)
    out_t = _make_embed_call()(norm_length.reshape(-1), classes,
                               jnp.asarray(_MF, dtype=jnp.bfloat16), w)
    return out_t.T


# speedup vs baseline: 1.1405x; 1.1405x over previous
import functools

import numpy as np
import jax
import jax.numpy as jnp
from jax import lax
from jax.experimental import pallas as pl
from jax.experimental.pallas import tpu as pltpu
from jax.experimental.pallas import tpu_sc as plsc

_NUM_TYPES = 4
_NUM_SPLINES = 16
_SPLINE_SPAN = 12
_NUM_CHANNELS = 64
_N_NODES = 10000
_N_EDGES = 160000


_NC = 2
_NS = 16
_NW = _NC * _NS
_NTILES = _N_EDGES // 128
_WTILES = 40
_WEDGES = _WTILES * 128
_WVECS = _WEDGES // 16


def _classes_body(types_hbm, ei_hbm, cls_hbm, table_v, ei_v, c_v):
    wid = lax.axis_index("s") * _NC + lax.axis_index("c")
    t0 = (wid * _NTILES) // _NW
    base = t0 * 128
    pltpu.sync_copy(types_hbm, table_v)
    pltpu.sync_copy(ei_hbm.at[:, pl.ds(base, _WEDGES)], ei_v)

    def body(i, carry):
        off = i * 16
        t0v = plsc.load_gather(table_v, [ei_v[0, pl.ds(off, 16)]])
        t1v = plsc.load_gather(table_v, [ei_v[1, pl.ds(off, 16)]])
        c_v[pl.ds(off, 16)] = t0v * _NUM_TYPES + t1v
        return carry

    lax.fori_loop(0, _WVECS, body, 0)
    pltpu.sync_copy(c_v, cls_hbm.at[pl.ds(base, _WEDGES)])


def _make_classes_call():
    mesh = plsc.VectorSubcoreMesh(core_axis_name="c", subcore_axis_name="s")
    return pl.kernel(
        _classes_body,
        mesh=mesh,
        compiler_params=pltpu.CompilerParams(needs_layout_passes=False),
        out_type=jax.ShapeDtypeStruct((_N_EDGES,), jnp.int32),
        scratch_types=[
            pltpu.VMEM((_N_NODES,), jnp.int32),
            pltpu.VMEM((2, _WEDGES), jnp.int32),
            pltpu.VMEM((_WEDGES,), jnp.int32),
        ],
    )


_B = 16000
_NB = _N_EDGES // _B
_NSC = _NUM_TYPES * _NUM_TYPES * _NUM_SPLINES
_WIDTH = _SPLINE_SPAN / _NUM_SPLINES

_FREQ = np.float32(np.pi / _WIDTH)

_J = np.arange(_NSC)
_SJ = _J % _NUM_SPLINES
_KJ = _J // _NUM_SPLINES
_CJ = _SJ / (_NUM_SPLINES - 1.0)
_CLS_BLOCK = 2.0 * (_KJ[:, None] == np.arange(_NUM_TYPES * _NUM_TYPES)[None, :])
_QN = np.arange(32)
_QM = (_QN - 1) >> 1
_LO = _QM - 10
_HI = _QM + 11 + ((_QN - 1) & 1)
_WIN_BLOCK = 2.0 * ((_SJ[:, None] >= _LO[None, :]) & (_SJ[:, None] <= _HI[None, :]))
_MF = np.concatenate([
    (0.5 * np.cos(np.pi * _CJ / _WIDTH))[:, None],
    (0.5 * np.sin(np.pi * _CJ / _WIDTH))[:, None],
    np.full((_NSC, 1), 0.5 - 4.0),
    _CLS_BLOCK,
    _WIN_BLOCK,
], axis=1).astype(np.float32)


def _embed_body(nl_ref, cls_ref, mf_ref, w_ref, out_ref):
    off = pl.multiple_of(pl.program_id(0) * _B, _B)
    x = nl_ref[pl.ds(off, _B)].reshape(1, _B)
    cls = cls_ref[pl.ds(off, _B)].reshape(1, _B)
    a = x * _FREQ
    ohc = (lax.broadcasted_iota(jnp.int32, (16, _B), 0) == cls
           ).astype(jnp.bfloat16)
    qn = jnp.floor(x * 30.0 - 0.5).astype(jnp.int32) + 1
    ohw = (lax.broadcasted_iota(jnp.int32, (32, _B), 0) == qn
           ).astype(jnp.bfloat16)
    rows = jnp.concatenate(
        [jnp.cos(a).astype(jnp.bfloat16), jnp.sin(a).astype(jnp.bfloat16),
         jnp.ones((1, _B), jnp.bfloat16), ohc, ohw], axis=0)
    combined = jnp.maximum(
        jnp.dot(mf_ref[...], rows, preferred_element_type=jnp.float32),
        0.0).astype(jnp.bfloat16)
    out_ref[...] = lax.dot_general(
        w_ref[...], combined, (((0,), (0,)), ((), ())),
        preferred_element_type=jnp.float32)


def _make_embed_call():
    return pl.pallas_call(
        _embed_body,
        grid=(_NB,),
        in_specs=[
            pl.BlockSpec((_N_EDGES,), lambda i: (0,)),
            pl.BlockSpec((_N_EDGES,), lambda i: (0,)),
            pl.BlockSpec((_NSC, 51), lambda i: (0, 0)),
            pl.BlockSpec((_NSC, _NUM_CHANNELS), lambda i: (0, 0)),
        ],
        out_specs=pl.BlockSpec((_NUM_CHANNELS, _B), lambda i: (0, i)),
        out_shape=jax.ShapeDtypeStruct((_NUM_CHANNELS, _N_EDGES), jnp.float32),
    )


def kernel(norm_length, atom_types, edge_index, class_embed_weight):
    at = atom_types.reshape(-1).astype(jnp.int32)
    classes = _make_classes_call()(at, edge_index.astype(jnp.int32))
    w = class_embed_weight.reshape(_NSC, _NUM_CHANNELS).astype(jnp.bfloat16)
    out_t = _make_embed_call()(norm_length.reshape(-1), classes,
                               jnp.asarray(_MF, dtype=jnp.bfloat16), w)
    return out_t.T


# speedup vs baseline: 1.1407x; 1.0002x over previous
import functools

import numpy as np
import jax
import jax.numpy as jnp
from jax import lax
from jax.experimental import pallas as pl
from jax.experimental.pallas import tpu as pltpu
from jax.experimental.pallas import tpu_sc as plsc

_NUM_TYPES = 4
_NUM_SPLINES = 16
_SPLINE_SPAN = 12
_NUM_CHANNELS = 64
_N_NODES = 10000
_N_EDGES = 160000


_NC = 2
_NS = 16
_NW = _NC * _NS
_NTILES = _N_EDGES // 128
_WTILES = 40
_WEDGES = _WTILES * 128
_WVECS = _WEDGES // 16


def _classes_body(types_hbm, ei_hbm, cls_hbm, table_v, ei_v, c_v):
    wid = lax.axis_index("s") * _NC + lax.axis_index("c")
    t0 = (wid * _NTILES) // _NW
    base = t0 * 128
    pltpu.sync_copy(types_hbm, table_v)
    pltpu.sync_copy(ei_hbm.at[:, pl.ds(base, _WEDGES)], ei_v)

    def body(i, carry):
        for u in range(8):
            off = i * 128 + u * 16
            t0v = plsc.load_gather(table_v, [ei_v[0, pl.ds(off, 16)]])
            t1v = plsc.load_gather(table_v, [ei_v[1, pl.ds(off, 16)]])
            c_v[pl.ds(off, 16)] = t0v * _NUM_TYPES + t1v
        return carry

    lax.fori_loop(0, _WVECS // 8, body, 0)
    pltpu.sync_copy(c_v, cls_hbm.at[pl.ds(base, _WEDGES)])


def _make_classes_call():
    mesh = plsc.VectorSubcoreMesh(core_axis_name="c", subcore_axis_name="s")
    return pl.kernel(
        _classes_body,
        mesh=mesh,
        compiler_params=pltpu.CompilerParams(needs_layout_passes=False),
        out_type=jax.ShapeDtypeStruct((_N_EDGES,), jnp.int32),
        scratch_types=[
            pltpu.VMEM((_N_NODES,), jnp.int32),
            pltpu.VMEM((2, _WEDGES), jnp.int32),
            pltpu.VMEM((_WEDGES,), jnp.int32),
        ],
    )


_B = 16000
_NB = _N_EDGES // _B
_NSC = _NUM_TYPES * _NUM_TYPES * _NUM_SPLINES
_WIDTH = _SPLINE_SPAN / _NUM_SPLINES

_FREQ = np.float32(np.pi / _WIDTH)

_J = np.arange(_NSC)
_SJ = _J % _NUM_SPLINES
_KJ = _J // _NUM_SPLINES
_CJ = _SJ / (_NUM_SPLINES - 1.0)
_CLS_BLOCK = 2.0 * (_KJ[:, None] == np.arange(_NUM_TYPES * _NUM_TYPES)[None, :])
_QN = np.arange(32)
_QM = (_QN - 1) >> 1
_LO = _QM - 10
_HI = _QM + 11 + ((_QN - 1) & 1)
_WIN_BLOCK = 2.0 * ((_SJ[:, None] >= _LO[None, :]) & (_SJ[:, None] <= _HI[None, :]))
_MF = np.concatenate([
    (0.5 * np.cos(np.pi * _CJ / _WIDTH))[:, None],
    (0.5 * np.sin(np.pi * _CJ / _WIDTH))[:, None],
    np.full((_NSC, 1), 0.5 - 4.0),
    _CLS_BLOCK,
    _WIN_BLOCK,
], axis=1).astype(np.float32)


def _embed_body(nl_ref, cls_ref, mf_ref, w_ref, out_ref):
    off = pl.multiple_of(pl.program_id(0) * _B, _B)
    x = nl_ref[pl.ds(off, _B)].reshape(1, _B)
    cls = cls_ref[pl.ds(off, _B)].reshape(1, _B)
    a = x * _FREQ
    ohc = (lax.broadcasted_iota(jnp.int32, (16, _B), 0) == cls
           ).astype(jnp.bfloat16)
    qn = jnp.floor(x * 30.0 - 0.5).astype(jnp.int32) + 1
    ohw = (lax.broadcasted_iota(jnp.int32, (32, _B), 0) == qn
           ).astype(jnp.bfloat16)
    rows = jnp.concatenate(
        [jnp.cos(a).astype(jnp.bfloat16), jnp.sin(a).astype(jnp.bfloat16),
         jnp.ones((1, _B), jnp.bfloat16), ohc, ohw], axis=0)
    combined = jnp.maximum(
        jnp.dot(mf_ref[...], rows, preferred_element_type=jnp.float32),
        0.0).astype(jnp.bfloat16)
    out_ref[...] = lax.dot_general(
        w_ref[...], combined, (((0,), (0,)), ((), ())),
        preferred_element_type=jnp.float32)


def _make_embed_call():
    return pl.pallas_call(
        _embed_body,
        grid=(_NB,),
        in_specs=[
            pl.BlockSpec((_N_EDGES,), lambda i: (0,)),
            pl.BlockSpec((_N_EDGES,), lambda i: (0,)),
            pl.BlockSpec((_NSC, 51), lambda i: (0, 0)),
            pl.BlockSpec((_NSC, _NUM_CHANNELS), lambda i: (0, 0)),
        ],
        out_specs=pl.BlockSpec((_NUM_CHANNELS, _B), lambda i: (0, i)),
        out_shape=jax.ShapeDtypeStruct((_NUM_CHANNELS, _N_EDGES), jnp.float32),
    )


def kernel(norm_length, atom_types, edge_index, class_embed_weight):
    at = atom_types.reshape(-1).astype(jnp.int32)
    classes = _make_classes_call()(at, edge_index.astype(jnp.int32))
    w = class_embed_weight.reshape(_NSC, _NUM_CHANNELS).astype(jnp.bfloat16)
    out_t = _make_embed_call()(norm_length.reshape(-1), classes,
                               jnp.asarray(_MF, dtype=jnp.bfloat16), w)
    return out_t.T


# speedup vs baseline: 1.1562x; 1.0136x over previous
import functools

import numpy as np
import jax
import jax.numpy as jnp
from jax import lax
from jax.experimental import pallas as pl
from jax.experimental.pallas import tpu as pltpu
from jax.experimental.pallas import tpu_sc as plsc

_NUM_TYPES = 4
_NUM_SPLINES = 16
_SPLINE_SPAN = 12
_NUM_CHANNELS = 64
_N_NODES = 10000
_N_EDGES = 160000


_NC = 2
_NS = 16
_NW = _NC * _NS
_NTILES = _N_EDGES // 128
_WTILES = 40
_WEDGES = _WTILES * 128
_WVECS = _WEDGES // 16


def _classes_body(types_hbm, ei_hbm, cls_hbm, table_v, ei_v, c_v, sem, sem2):
    wid = lax.axis_index("s") * _NC + lax.axis_index("c")
    t0 = (wid * _NTILES) // _NW
    base = t0 * 128
    cp1 = pltpu.async_copy(types_hbm, table_v, sem)
    cp2 = pltpu.async_copy(ei_hbm.at[:, pl.ds(base, _WEDGES)], ei_v, sem2)
    cp1.wait()
    cp2.wait()

    def body(i, carry):
        for u in range(8):
            off = i * 128 + u * 16
            t0v = plsc.load_gather(table_v, [ei_v[0, pl.ds(off, 16)]])
            t1v = plsc.load_gather(table_v, [ei_v[1, pl.ds(off, 16)]])
            c_v[pl.ds(off, 16)] = t0v * _NUM_TYPES + t1v
        return carry

    lax.fori_loop(0, _WVECS // 8, body, 0)
    pltpu.sync_copy(c_v, cls_hbm.at[pl.ds(base, _WEDGES)])


def _make_classes_call():
    mesh = plsc.VectorSubcoreMesh(core_axis_name="c", subcore_axis_name="s")
    return pl.kernel(
        _classes_body,
        mesh=mesh,
        compiler_params=pltpu.CompilerParams(needs_layout_passes=False),
        out_type=jax.ShapeDtypeStruct((_N_EDGES,), jnp.int32),
        scratch_types=[
            pltpu.VMEM((_N_NODES,), jnp.int32),
            pltpu.VMEM((2, _WEDGES), jnp.int32),
            pltpu.VMEM((_WEDGES,), jnp.int32),
            pltpu.SemaphoreType.DMA,
            pltpu.SemaphoreType.DMA,
        ],
    )


_B = 16000
_NB = _N_EDGES // _B
_NSC = _NUM_TYPES * _NUM_TYPES * _NUM_SPLINES
_WIDTH = _SPLINE_SPAN / _NUM_SPLINES

_FREQ = np.float32(np.pi / _WIDTH)

_J = np.arange(_NSC)
_SJ = _J % _NUM_SPLINES
_KJ = _J // _NUM_SPLINES
_CJ = _SJ / (_NUM_SPLINES - 1.0)
_CLS_BLOCK = 2.0 * (_KJ[:, None] == np.arange(_NUM_TYPES * _NUM_TYPES)[None, :])
_QN = np.arange(32)
_QM = (_QN - 1) >> 1
_LO = _QM - 10
_HI = _QM + 11 + ((_QN - 1) & 1)
_WIN_BLOCK = 2.0 * ((_SJ[:, None] >= _LO[None, :]) & (_SJ[:, None] <= _HI[None, :]))
_MF = np.concatenate([
    (0.5 * np.cos(np.pi * _CJ / _WIDTH))[:, None],
    (0.5 * np.sin(np.pi * _CJ / _WIDTH))[:, None],
    np.full((_NSC, 1), 0.5 - 4.0),
    _CLS_BLOCK,
    _WIN_BLOCK,
], axis=1).astype(np.float32)


def _embed_body(nl_ref, cls_ref, mf_ref, w_ref, out_ref):
    off = pl.multiple_of(pl.program_id(0) * _B, _B)
    x = nl_ref[pl.ds(off, _B)].reshape(1, _B)
    cls = cls_ref[pl.ds(off, _B)].reshape(1, _B)
    a = x * _FREQ
    ohc = (lax.broadcasted_iota(jnp.int32, (16, _B), 0) == cls
           ).astype(jnp.bfloat16)
    qn = jnp.floor(x * 30.0 - 0.5).astype(jnp.int32) + 1
    ohw = (lax.broadcasted_iota(jnp.int32, (32, _B), 0) == qn
           ).astype(jnp.bfloat16)
    rows = jnp.concatenate(
        [jnp.cos(a).astype(jnp.bfloat16), jnp.sin(a).astype(jnp.bfloat16),
         jnp.ones((1, _B), jnp.bfloat16), ohc, ohw], axis=0)
    combined = jnp.maximum(
        jnp.dot(mf_ref[...], rows, preferred_element_type=jnp.float32),
        0.0).astype(jnp.bfloat16)
    out_ref[...] = lax.dot_general(
        w_ref[...], combined, (((0,), (0,)), ((), ())),
        preferred_element_type=jnp.float32)


def _make_embed_call():
    return pl.pallas_call(
        _embed_body,
        grid=(_NB,),
        in_specs=[
            pl.BlockSpec((_N_EDGES,), lambda i: (0,)),
            pl.BlockSpec((_N_EDGES,), lambda i: (0,)),
            pl.BlockSpec((_NSC, 51), lambda i: (0, 0)),
            pl.BlockSpec((_NSC, _NUM_CHANNELS), lambda i: (0, 0)),
        ],
        out_specs=pl.BlockSpec((_NUM_CHANNELS, _B), lambda i: (0, i)),
        out_shape=jax.ShapeDtypeStruct((_NUM_CHANNELS, _N_EDGES), jnp.float32),
    )


def kernel(norm_length, atom_types, edge_index, class_embed_weight):
    at = atom_types.reshape(-1).astype(jnp.int32)
    classes = _make_classes_call()(at, edge_index.astype(jnp.int32))
    w = class_embed_weight.reshape(_NSC, _NUM_CHANNELS).astype(jnp.bfloat16)
    out_t = _make_embed_call()(norm_length.reshape(-1), classes,
                               jnp.asarray(_MF, dtype=jnp.bfloat16), w)
    return out_t.T
